# baseline (speedup 1.0000x reference)
import jax
import jax.numpy as jnp
import numpy as np
from jax.experimental import pallas as pl
from jax.experimental.pallas import tpu as pltpu

_L = 512
_D = 768
_WD = 64
_F = 2 * _D + _WD
_B = 8
_NW = 8
_ST = 2048
_NT = 2

_BUCKET_BINS = [0, 1, 2, 3, 4, 5, 7, 8, 15, 16, 31, 32, 63, 64]


def _bucket_of(width):
    return max(ix for ix, v in enumerate(_BUCKET_BINS) if width >= v)


def _span_index_consts():
    starts, ends = [], []
    for w in range(1, _NW + 1):
        for i in range(0, _L - w + 1):
            starts.append(i)
            ends.append(i + w - 1)
    return (np.array(starts, dtype=np.int32), np.array(ends, dtype=np.int32))


_STARTS_NP, _ENDS_NP = _span_index_consts()
_NS = _STARTS_NP.shape[0]
_OFFS = [0]
for _w in range(1, _NW + 1):
    _OFFS.append(_OFFS[-1] + (_L - _w + 1))


def _tile_segments(t):
    lo, hi = _ST * t, min(_ST * (t + 1), _NS)
    segs = []
    for w in range(1, _NW + 1):
        s_lo, s_hi = max(lo, _OFFS[w - 1]), min(hi, _OFFS[w])
        if s_lo < s_hi:
            segs.append((w, s_lo, s_hi))
    return segs


def _tc_body(x_ref, xnext_ref, wt_ref, out_ref, xt_ref):
    b = pl.program_id(0)
    t = pl.program_id(1)
    par = jax.lax.rem(b, 2)

    @pl.when((b == 0) & (t == 0))
    def _():
        xt_ref[0] = jnp.swapaxes(x_ref[0], 0, 1)

    @pl.when((t == _NT - 1) & (b < _B - 1))
    def _():
        xt_ref[1 - par] = jnp.swapaxes(xnext_ref[0], 0, 1)

    for tt in range(_NT):
        @pl.when(t == tt)
        def _(tt=tt):
            for w, s_lo, s_hi in _tile_segments(tt):
                a = s_lo - _ST * tt
                c = s_hi - s_lo
                s0 = s_lo - _OFFS[w - 1]
                out_ref[0, 0:_D, a:a + c] = xt_ref[par, :, s0:s0 + c]
                out_ref[0, _D:2 * _D, a:a + c] = (
                    xt_ref[par, :, s0 + w - 1:s0 + w - 1 + c])
                wrow = wt_ref[_bucket_of(w), :]
                out_ref[0, 2 * _D:_F, a:a + c] = jnp.broadcast_to(
                    wrow[:, None], (_WD, c))


def kernel(x, width_table, batch_max_seq_len):
    spans_t = pl.pallas_call(
        _tc_body,
        grid=(_B, _NT),
        in_specs=[
            pl.BlockSpec((1, _L, _D), lambda b, t: (b, 0, 0)),
            pl.BlockSpec(
                (1, _L, _D),
                lambda b, t: (jnp.minimum(b + 1, _B - 1), 0, 0)),
            pl.BlockSpec((14, _WD), lambda b, t: (0, 0)),
        ],
        out_specs=pl.BlockSpec((1, _F, _ST), lambda b, t: (b, 0, t)),
        out_shape=jax.ShapeDtypeStruct((_B, _F, _NS), jnp.float32),
        scratch_shapes=[pltpu.VMEM((2, _D, _L), jnp.float32)],
    )(x, x, width_table)
    spans = jnp.swapaxes(spans_t, 1, 2)
    residual = jnp.asarray(batch_max_seq_len, jnp.int32) - jnp.int32(_L)
    span_indices = jnp.stack(
        [jnp.asarray(_STARTS_NP) + residual, jnp.asarray(_ENDS_NP)], axis=1)
    return (spans, span_indices)


# speedup vs baseline: 1.0501x; 1.0501x over previous
import jax
import jax.numpy as jnp
import numpy as np
from jax.experimental import pallas as pl
from jax.experimental.pallas import tpu as pltpu

_L = 512
_D = 768
_WD = 64
_F = 2 * _D + _WD
_B = 8
_NW = 8
_ST = 2048
_NT = 2

_BUCKET_BINS = [0, 1, 2, 3, 4, 5, 7, 8, 15, 16, 31, 32, 63, 64]


def _bucket_of(width):
    return max(ix for ix, v in enumerate(_BUCKET_BINS) if width >= v)


def _span_index_consts():
    starts, ends = [], []
    for w in range(1, _NW + 1):
        for i in range(0, _L - w + 1):
            starts.append(i)
            ends.append(i + w - 1)
    return (np.array(starts, dtype=np.int32), np.array(ends, dtype=np.int32))


_STARTS_NP, _ENDS_NP = _span_index_consts()
_NS = _STARTS_NP.shape[0]
_OFFS = [0]
for _w in range(1, _NW + 1):
    _OFFS.append(_OFFS[-1] + (_L - _w + 1))


def _tile_segments(t):
    lo, hi = _ST * t, min(_ST * (t + 1), _NS)
    segs = []
    for w in range(1, _NW + 1):
        s_lo, s_hi = max(lo, _OFFS[w - 1]), min(hi, _OFFS[w])
        if s_lo < s_hi:
            segs.append((w, s_lo, s_hi))
    return segs


def _tc_body(x_ref, wt_ref, out_ref, xt_ref):
    t = pl.program_id(1)

    @pl.when(t == 0)
    def _():
        xt_ref[...] = jnp.swapaxes(x_ref[0], 0, 1)

    for tt in range(_NT):
        @pl.when(t == tt)
        def _(tt=tt):
            for w, s_lo, s_hi in _tile_segments(tt):
                a = s_lo - _ST * tt
                c = s_hi - s_lo
                s0 = s_lo - _OFFS[w - 1]
                out_ref[0, 0:_D, a:a + c] = xt_ref[:, s0:s0 + c]
                out_ref[0, _D:2 * _D, a:a + c] = (
                    xt_ref[:, s0 + w - 1:s0 + w - 1 + c])
                wrow = wt_ref[_bucket_of(w), :]
                out_ref[0, 2 * _D:_F, a:a + c] = jnp.broadcast_to(
                    wrow[:, None], (_WD, c))


def kernel(x, width_table, batch_max_seq_len):
    spans_t = pl.pallas_call(
        _tc_body,
        grid=(_B, _NT),
        in_specs=[
            pl.BlockSpec((1, _L, _D), lambda b, t: (b, 0, 0)),
            pl.BlockSpec((14, _WD), lambda b, t: (0, 0)),
        ],
        out_specs=pl.BlockSpec((1, _F, _ST), lambda b, t: (b, 0, t)),
        out_shape=jax.ShapeDtypeStruct((_B, _F, _NS), jnp.float32),
        scratch_shapes=[pltpu.VMEM((_D, _L), jnp.float32)],
    )(x, width_table)
    spans = jnp.swapaxes(spans_t, 1, 2)
    residual = jnp.asarray(batch_max_seq_len, jnp.int32) - jnp.int32(_L)
    span_indices = jnp.stack(
        [jnp.asarray(_STARTS_NP) + residual, jnp.asarray(_ENDS_NP)], axis=1)
    return (spans, span_indices)
